# baseline (speedup 1.0000x reference)
import functools

import jax
import jax.numpy as jnp
from jax import lax
from jax.experimental import pallas as pl
from jax.experimental.pallas import tpu as pltpu
from jax.experimental.pallas import tpu_sc as plsc

N_TOK = 32768
N_CLS = 128
BLK = 1024
N_BLKS = N_TOK // BLK
NVEC_ALL = N_TOK // 16


def _stage1_body(x_ref, t_ref, ce_ref, acc_ref, ces_ref):
    i = pl.program_id(0)
    x = x_ref[0]
    t = t_ref[0]
    m = jnp.max(x, axis=2)
    e = jnp.exp(x - m[:, :, None])
    s = jnp.sum(e, axis=2)
    lse = jnp.log(s) + m
    cls = lax.broadcasted_iota(jnp.int32, (8, 128, N_CLS), 2)
    gold = jnp.sum(jnp.where(cls == t[:, :, None], x, 0.0), axis=2)
    ce = lse - gold
    is_pos = t > 0
    tf = t.astype(jnp.float32)
    pt = jnp.exp(-ce)
    focal = (1.0 - pt) * (1.0 - pt) * ce
    n_pos_blk = jnp.sum(tf)
    s_pos_blk = jnp.sum(jnp.where(is_pos, focal, 0.0))
    s_negall_blk = jnp.sum(jnp.where(is_pos, 0.0, 0.25 * focal))
    ce_neg = jnp.where(is_pos, -1.0, ce)
    ce_ref[0] = ce_neg
    ces_ref[pl.ds(i * 8, 8), :] = ce_neg

    lanes = lax.broadcasted_iota(jnp.int32, (1, 128), 1)
    upd = (jnp.where(lanes == 0, n_pos_blk, 0.0)
           + jnp.where(lanes == 1, s_pos_blk, 0.0)
           + jnp.where(lanes == 2, s_negall_blk, 0.0))

    @pl.when(i == 0)
    def _():
        acc_ref[...] = jnp.zeros_like(acc_ref)

    acc_ref[...] += upd

    @pl.when(i == N_BLKS - 1)
    def _():
        v = ces_ref[...]
        key = jnp.maximum(lax.bitcast_convert_type(v, jnp.int32), 0)
        n_pos = acc_ref[0, 0].astype(jnp.int32)
        n_neg = N_TOK - n_pos
        k = jnp.minimum(n_pos // 4, n_neg)

        def bit_step(j, prefix):
            cand = prefix | (1 << (30 - j))
            cnt = jnp.sum((key >= cand).astype(jnp.int32))
            return jnp.where(cnt >= k, cand, prefix)

        prefix = lax.fori_loop(0, 31, bit_step, jnp.int32(0))
        tau_bits = jnp.where(k > 0, prefix, jnp.int32(0x7F800000))
        tau_f = lax.bitcast_convert_type(tau_bits, jnp.float32)
        acc_ref[...] += (jnp.where(lanes == 3, tau_f, 0.0)
                         + jnp.where(lanes == 4, k.astype(jnp.float32), 0.0))


_sc_mesh = plsc.VectorSubcoreMesh(
    core_axis_name="c", subcore_axis_name="s", num_cores=1)

_GDN = lax.GatherDimensionNumbers(
    offset_dims=(), collapsed_slice_dims=(0,), start_index_map=(0,))


def _take16(v, idx):
    return lax.gather(v, idx[:, None], _GDN, (1,),
                      mode=lax.GatherScatterMode.PROMISE_IN_BOUNDS)


@functools.partial(
    pl.kernel,
    out_type=jax.ShapeDtypeStruct((16,), jnp.float32),
    mesh=_sc_mesh,
    scratch_types=[
        pltpu.VMEM((N_TOK,), jnp.float32),
        pltpu.VMEM((16,), jnp.float32),
        pltpu.VMEM((16,), jnp.float32),
    ],
)
def _stage2_sc(nc_hbm, scal_hbm, out_hbm, vals, scalv, stf):
    sid = lax.axis_index("s")

    @pl.when(sid == 0)
    def _():
        pltpu.sync_copy(nc_hbm, vals)
        pltpu.sync_copy(scal_hbm, scalv)
        lanes = lax.iota(jnp.int32, 16)
        sv = scalv[...]
        n_pos = sv[0].astype(jnp.int32)
        s_pos = sv[1]
        s_negall = sv[2]
        k = sv[4].astype(jnp.int32)
        tau_v = jnp.broadcast_to(sv[3], (16,))
        tau_bits_v = lax.bitcast_convert_type(tau_v, jnp.int32)

        zf = jnp.zeros((16,), jnp.float32)

        def fbody(j, acc2):
            accs, accc = acc2
            v = vals[pl.ds(j * 16, 16)]
            kb = jnp.maximum(lax.bitcast_convert_type(v, jnp.int32), 0)
            gt = kb > tau_bits_v
            e = jnp.exp(-v)
            fl = 0.25 * (1.0 - e) * (1.0 - e) * v
            return (accs + jnp.where(gt, fl, 0.0),
                    accc + jnp.where(gt, 1.0, 0.0))
        accs, accc = lax.fori_loop(0, NVEC_ALL, fbody, (zf, zf))
        for sh in (8, 4, 2, 1):
            accs = accs + _take16(accs, lanes ^ sh)
            accc = accc + _take16(accc, lanes ^ sh)
        s_gt = jnp.broadcast_to(accs[0], (16,))
        cnt_gt = jnp.broadcast_to(accc[0], (16,))

        ev = jnp.exp(-tau_v)
        g_tau = 0.25 * (1.0 - ev) * (1.0 - ev) * tau_v
        kf = k.astype(jnp.float32)
        s_hard = jnp.where(k > 0, s_gt + (kf - cnt_gt) * g_tau, 0.0)
        s_pos_v = jnp.broadcast_to(s_pos, (16,))
        n_sel_v = jnp.broadcast_to((n_pos + k).astype(jnp.float32), (16,))
        sel_mean = (s_pos_v + s_hard) / jnp.maximum(n_sel_v, 1.0)
        full_mean = (s_pos_v + jnp.broadcast_to(s_negall, (16,))) \
            * jnp.float32(1.0 / N_TOK)
        res = jnp.where(n_pos == 0, full_mean, sel_mean)
        stf[...] = jnp.where(lanes == 0, res, 0.0)
        pltpu.sync_copy(stf, out_hbm)


@jax.jit
def kernel(inputs, targets):
    x4 = inputs.reshape(N_BLKS, 8, 128, N_CLS)
    t3 = targets.reshape(N_BLKS, 8, 128)

    neg_ce, acc = pl.pallas_call(
        _stage1_body,
        grid=(N_BLKS,),
        in_specs=[
            pl.BlockSpec((1, 8, 128, N_CLS), lambda i: (i, 0, 0, 0)),
            pl.BlockSpec((1, 8, 128), lambda i: (i, 0, 0)),
        ],
        out_specs=[
            pl.BlockSpec((1, 8, 128), lambda i: (i, 0, 0)),
            pl.BlockSpec((1, 128), lambda i: (0, 0)),
        ],
        out_shape=[
            jax.ShapeDtypeStruct((N_BLKS, 8, 128), jnp.float32),
            jax.ShapeDtypeStruct((1, 128), jnp.float32),
        ],
        scratch_shapes=[pltpu.VMEM((N_TOK // 128, 128), jnp.float32)],
    )(x4, t3)

    scal16 = acc.reshape(128)[:16]
    nc_flat = neg_ce.reshape(N_TOK)

    out16 = _stage2_sc(nc_flat, scal16)
    return out16[0]


# speedup vs baseline: 1.0231x; 1.0231x over previous
import functools

import jax
import jax.numpy as jnp
from jax import lax
from jax.experimental import pallas as pl
from jax.experimental.pallas import tpu as pltpu
from jax.experimental.pallas import tpu_sc as plsc

N_TOK = 32768
N_CLS = 128
BLK = 1024
N_BLKS = N_TOK // BLK
NVEC_ALL = N_TOK // 16


def _stage1_body(x_ref, t_ref, ce_ref, acc_ref, ces_ref):
    i = pl.program_id(0)
    x = x_ref[0]
    t = t_ref[0]
    m = jnp.max(x, axis=2)
    e = jnp.exp(x - m[:, :, None])
    s = jnp.sum(e, axis=2)
    lse = jnp.log(s) + m
    cls = lax.broadcasted_iota(jnp.int32, (8, 128, N_CLS), 2)
    gold = jnp.sum(jnp.where(cls == t[:, :, None], x, 0.0), axis=2)
    ce = lse - gold
    is_pos = t > 0
    tf = t.astype(jnp.float32)
    pt = jnp.exp(-ce)
    focal = (1.0 - pt) * (1.0 - pt) * ce
    n_pos_blk = jnp.sum(tf)
    s_pos_blk = jnp.sum(jnp.where(is_pos, focal, 0.0))
    s_negall_blk = jnp.sum(jnp.where(is_pos, 0.0, 0.25 * focal))
    ce_neg = jnp.where(is_pos, -1.0, ce)
    ce_ref[0] = ce_neg
    ces_ref[pl.ds(i * 8, 8), :] = ce_neg

    lanes = lax.broadcasted_iota(jnp.int32, (1, 128), 1)
    upd = (jnp.where(lanes == 0, n_pos_blk, 0.0)
           + jnp.where(lanes == 1, s_pos_blk, 0.0)
           + jnp.where(lanes == 2, s_negall_blk, 0.0))

    @pl.when(i == 0)
    def _():
        acc_ref[...] = jnp.zeros_like(acc_ref)

    acc_ref[...] += upd

    @pl.when(i == N_BLKS - 1)
    def _():
        v = ces_ref[...]
        key = jnp.maximum(lax.bitcast_convert_type(v, jnp.int32), 0)
        n_pos = acc_ref[0, 0].astype(jnp.int32)
        n_neg = N_TOK - n_pos
        k = jnp.minimum(n_pos // 4, n_neg)

        def bit_step(j, prefix):
            cand = prefix | (1 << (30 - j))
            cnt = jnp.sum((key >= cand).astype(jnp.int32))
            return jnp.where(cnt >= k, cand, prefix)

        prefix = lax.fori_loop(0, 31, bit_step, jnp.int32(0))
        tau_bits = jnp.where(k > 0, prefix, jnp.int32(0x7F800000))
        tau_f = lax.bitcast_convert_type(tau_bits, jnp.float32)
        acc_ref[...] += (jnp.where(lanes == 3, tau_f, 0.0)
                         + jnp.where(lanes == 4, k.astype(jnp.float32), 0.0))


_sc_mesh = plsc.VectorSubcoreMesh(
    core_axis_name="c", subcore_axis_name="s", num_cores=1)

_GDN = lax.GatherDimensionNumbers(
    offset_dims=(), collapsed_slice_dims=(0,), start_index_map=(0,))


def _take16(v, idx):
    return lax.gather(v, idx[:, None], _GDN, (1,),
                      mode=lax.GatherScatterMode.PROMISE_IN_BOUNDS)


@functools.partial(
    pl.kernel,
    out_type=jax.ShapeDtypeStruct((16,), jnp.float32),
    mesh=_sc_mesh,
    scratch_types=[
        pltpu.VMEM((N_TOK,), jnp.float32),
        pltpu.VMEM((16,), jnp.float32),
        pltpu.VMEM((16,), jnp.float32),
    ],
)
def _stage2_sc(nc_hbm, scal_hbm, out_hbm, vals, scalv, stf):
    sid = lax.axis_index("s")

    @pl.when(sid == 0)
    def _():
        pltpu.sync_copy(nc_hbm, vals)
        pltpu.sync_copy(scal_hbm, scalv)
        lanes = lax.iota(jnp.int32, 16)
        sv = scalv[...]
        n_pos = sv[0].astype(jnp.int32)
        s_pos = sv[1]
        s_negall = sv[2]
        k = sv[4].astype(jnp.int32)
        tau_v = jnp.broadcast_to(sv[3], (16,))
        tau_bits_v = lax.bitcast_convert_type(tau_v, jnp.int32)

        zf = jnp.zeros((16,), jnp.float32)

        def fbody(j, acc2):
            accs, accc = acc2
            for u in range(4):
                v = vals[pl.ds(j * 64 + u * 16, 16)]
                kb = jnp.maximum(lax.bitcast_convert_type(v, jnp.int32), 0)
                gt = kb > tau_bits_v
                e = jnp.exp(-v)
                fl = 0.25 * (1.0 - e) * (1.0 - e) * v
                accs = accs + jnp.where(gt, fl, 0.0)
                accc = accc + jnp.where(gt, 1.0, 0.0)
            return (accs, accc)
        accs, accc = lax.fori_loop(0, NVEC_ALL // 4, fbody, (zf, zf))
        for sh in (8, 4, 2, 1):
            accs = accs + _take16(accs, lanes ^ sh)
            accc = accc + _take16(accc, lanes ^ sh)
        s_gt = jnp.broadcast_to(accs[0], (16,))
        cnt_gt = jnp.broadcast_to(accc[0], (16,))

        ev = jnp.exp(-tau_v)
        g_tau = 0.25 * (1.0 - ev) * (1.0 - ev) * tau_v
        kf = k.astype(jnp.float32)
        s_hard = jnp.where(k > 0, s_gt + (kf - cnt_gt) * g_tau, 0.0)
        s_pos_v = jnp.broadcast_to(s_pos, (16,))
        n_sel_v = jnp.broadcast_to((n_pos + k).astype(jnp.float32), (16,))
        sel_mean = (s_pos_v + s_hard) / jnp.maximum(n_sel_v, 1.0)
        full_mean = (s_pos_v + jnp.broadcast_to(s_negall, (16,))) \
            * jnp.float32(1.0 / N_TOK)
        res = jnp.where(n_pos == 0, full_mean, sel_mean)
        stf[...] = jnp.where(lanes == 0, res, 0.0)
        pltpu.sync_copy(stf, out_hbm)


@jax.jit
def kernel(inputs, targets):
    x4 = inputs.reshape(N_BLKS, 8, 128, N_CLS)
    t3 = targets.reshape(N_BLKS, 8, 128)

    neg_ce, acc = pl.pallas_call(
        _stage1_body,
        grid=(N_BLKS,),
        in_specs=[
            pl.BlockSpec((1, 8, 128, N_CLS), lambda i: (i, 0, 0, 0)),
            pl.BlockSpec((1, 8, 128), lambda i: (i, 0, 0)),
        ],
        out_specs=[
            pl.BlockSpec((1, 8, 128), lambda i: (i, 0, 0)),
            pl.BlockSpec((1, 128), lambda i: (0, 0)),
        ],
        out_shape=[
            jax.ShapeDtypeStruct((N_BLKS, 8, 128), jnp.float32),
            jax.ShapeDtypeStruct((1, 128), jnp.float32),
        ],
        scratch_shapes=[pltpu.VMEM((N_TOK // 128, 128), jnp.float32)],
    )(x4, t3)

    scal16 = acc.reshape(128)[:16]
    nc_flat = neg_ce.reshape(N_TOK)

    out16 = _stage2_sc(nc_flat, scal16)
    return out16[0]


# speedup vs baseline: 1.0780x; 1.0537x over previous
import functools

import jax
import jax.numpy as jnp
from jax import lax
from jax.experimental import pallas as pl
from jax.experimental.pallas import tpu as pltpu
from jax.experimental.pallas import tpu_sc as plsc

N_TOK = 32768
N_CLS = 128
BLK = 1024
N_BLKS = N_TOK // BLK
NVEC_ALL = N_TOK // 16


def _stage1_body(x_ref, t_ref, ce_ref, acc_ref, ces_ref):
    i = pl.program_id(0)
    x = x_ref[0]
    t = t_ref[0]
    m = jnp.max(x, axis=2)
    e = jnp.exp(x - m[:, :, None])
    s = jnp.sum(e, axis=2)
    lse = jnp.log(s) + m
    cls = lax.broadcasted_iota(jnp.int32, (8, 128, N_CLS), 2)
    gold = jnp.sum(jnp.where(cls == t[:, :, None], x, 0.0), axis=2)
    ce = lse - gold
    is_pos = t > 0
    tf = t.astype(jnp.float32)
    pt = jnp.exp(-ce)
    focal = (1.0 - pt) * (1.0 - pt) * ce
    n_pos_blk = jnp.sum(tf)
    s_pos_blk = jnp.sum(jnp.where(is_pos, focal, 0.0))
    s_negall_blk = jnp.sum(jnp.where(is_pos, 0.0, 0.25 * focal))
    ce_neg = jnp.where(is_pos, -1.0, ce)
    ce_ref[0] = ce_neg
    ces_ref[pl.ds(i * 8, 8), :] = ce_neg

    lanes = lax.broadcasted_iota(jnp.int32, (1, 128), 1)
    upd = (jnp.where(lanes == 0, n_pos_blk, 0.0)
           + jnp.where(lanes == 1, s_pos_blk, 0.0)
           + jnp.where(lanes == 2, s_negall_blk, 0.0))

    @pl.when(i == 0)
    def _():
        acc_ref[...] = jnp.zeros_like(acc_ref)

    acc_ref[...] += upd

    @pl.when(i == N_BLKS - 1)
    def _():
        v = ces_ref[...]
        key = jnp.maximum(lax.bitcast_convert_type(v, jnp.int32), 0)
        n_pos = acc_ref[0, 0].astype(jnp.int32)
        n_neg = N_TOK - n_pos
        k = jnp.minimum(n_pos // 4, n_neg)

        def bit_step(j, prefix):
            cand = prefix | (1 << (30 - j))
            cnt = jnp.sum((key >= cand).astype(jnp.int32))
            return jnp.where(cnt >= k, cand, prefix)

        prefix = lax.fori_loop(0, 31, bit_step, jnp.int32(0))
        tau_bits = jnp.where(k > 0, prefix, jnp.int32(0x7F800000))
        tau_f = lax.bitcast_convert_type(tau_bits, jnp.float32)
        acc_ref[...] += (jnp.where(lanes == 3, tau_f, 0.0)
                         + jnp.where(lanes == 4, k.astype(jnp.float32), 0.0))


_sc_mesh = plsc.VectorSubcoreMesh(
    core_axis_name="c", subcore_axis_name="s", num_cores=1)

_GDN = lax.GatherDimensionNumbers(
    offset_dims=(), collapsed_slice_dims=(0,), start_index_map=(0,))


def _take16(v, idx):
    return lax.gather(v, idx[:, None], _GDN, (1,),
                      mode=lax.GatherScatterMode.PROMISE_IN_BOUNDS)


NW = 16
CHUNK = N_TOK // NW


@functools.partial(
    pl.kernel,
    out_type=jax.ShapeDtypeStruct((NW, 16), jnp.float32),
    mesh=_sc_mesh,
    scratch_types=[
        pltpu.VMEM((CHUNK,), jnp.float32),
        pltpu.VMEM((16,), jnp.float32),
        pltpu.VMEM((16,), jnp.float32),
    ],
)
def _stage2_sc(nc_hbm, scal_hbm, out_hbm, vals, scalv, stf):
    sid = lax.axis_index("s")
    pltpu.sync_copy(nc_hbm.at[pl.ds(sid * CHUNK, CHUNK)], vals)
    pltpu.sync_copy(scal_hbm, scalv)
    lanes = lax.iota(jnp.int32, 16)
    sv = scalv[...]
    tau_v = jnp.broadcast_to(sv[3], (16,))
    tau_bits_v = lax.bitcast_convert_type(tau_v, jnp.int32)

    zf = jnp.zeros((16,), jnp.float32)

    def fbody(j, acc2):
        accs, accc = acc2
        for u in range(4):
            v = vals[pl.ds(j * 64 + u * 16, 16)]
            kb = jnp.maximum(lax.bitcast_convert_type(v, jnp.int32), 0)
            gt = kb > tau_bits_v
            e = jnp.exp(-v)
            fl = 0.25 * (1.0 - e) * (1.0 - e) * v
            accs = accs + jnp.where(gt, fl, 0.0)
            accc = accc + jnp.where(gt, 1.0, 0.0)
        return (accs, accc)
    accs, accc = lax.fori_loop(0, CHUNK // 64, fbody, (zf, zf))
    for sh in (8, 4, 2, 1):
        accs = accs + _take16(accs, lanes ^ sh)
        accc = accc + _take16(accc, lanes ^ sh)
    part = jnp.where(lanes == 0, accs, jnp.where(lanes == 1, accc, 0.0))
    stf[...] = part
    pltpu.sync_copy(stf, out_hbm.at[sid])


def _merge_body(scal_ref, part_ref, out_ref):
    s_gt = jnp.float32(0.0)
    cnt_gt = jnp.float32(0.0)
    for t in range(NW):
        s_gt = s_gt + part_ref[t * 16]
        cnt_gt = cnt_gt + part_ref[t * 16 + 1]
    n_pos = scal_ref[0].astype(jnp.int32)
    s_pos = scal_ref[1]
    s_negall = scal_ref[2]
    tau = scal_ref[3]
    k = scal_ref[4].astype(jnp.int32)
    ev = jnp.exp(-tau)
    g_tau = 0.25 * (1.0 - ev) * (1.0 - ev) * tau
    kf = k.astype(jnp.float32)
    s_hard = jnp.where(k > 0, s_gt + (kf - cnt_gt) * g_tau, 0.0)
    n_sel_f = (n_pos + k).astype(jnp.float32)
    sel_mean = (s_pos + s_hard) / jnp.maximum(n_sel_f, 1.0)
    full_mean = (s_pos + s_negall) * jnp.float32(1.0 / N_TOK)
    res = jnp.where(n_pos == 0, full_mean, sel_mean)
    out_ref[...] = jnp.broadcast_to(res, (1, 1))


@jax.jit
def kernel(inputs, targets):
    x4 = inputs.reshape(N_BLKS, 8, 128, N_CLS)
    t3 = targets.reshape(N_BLKS, 8, 128)

    neg_ce, acc = pl.pallas_call(
        _stage1_body,
        grid=(N_BLKS,),
        in_specs=[
            pl.BlockSpec((1, 8, 128, N_CLS), lambda i: (i, 0, 0, 0)),
            pl.BlockSpec((1, 8, 128), lambda i: (i, 0, 0)),
        ],
        out_specs=[
            pl.BlockSpec((1, 8, 128), lambda i: (i, 0, 0)),
            pl.BlockSpec((1, 128), lambda i: (0, 0)),
        ],
        out_shape=[
            jax.ShapeDtypeStruct((N_BLKS, 8, 128), jnp.float32),
            jax.ShapeDtypeStruct((1, 128), jnp.float32),
        ],
        scratch_shapes=[pltpu.VMEM((N_TOK // 128, 128), jnp.float32)],
    )(x4, t3)

    scal16 = acc.reshape(128)[:16]
    nc_flat = neg_ce.reshape(N_TOK)

    parts = _stage2_sc(nc_flat, scal16)

    out = pl.pallas_call(
        _merge_body,
        in_specs=[
            pl.BlockSpec(memory_space=pltpu.SMEM),
            pl.BlockSpec(memory_space=pltpu.SMEM),
        ],
        out_specs=pl.BlockSpec((1, 1), lambda: (0, 0)),
        out_shape=jax.ShapeDtypeStruct((1, 1), jnp.float32),
    )(scal16, parts.reshape(NW * 16))
    return out[0, 0]
